# baseline (speedup 1.0000x reference)
import functools

import jax
import jax.numpy as jnp
from jax import lax
from jax.experimental import pallas as pl
from jax.experimental.pallas import tpu as pltpu
from jax.experimental.pallas import tpu_sc as plsc

B = 2
C = 96
H = 384
W = 384
N = H * W
NC = 2
NS = 16
NW = NC * NS
HALF = N // 2
NTASK = B * (C + 1) * 2
KMAX = (NTASK + NW - 1) // NW
WIN = 4 * W
NWIN = N // WIN
WPW = (B * NWIN) // NW
L = 16


def _pass1_body(flow_hbm, metric_hbm, dst_hbm, wgt_hbm,
                fxb, fyb, mb, dl0, dl1, wl0, wl1):
    wid = lax.axis_index("s") * NC + lax.axis_index("c")
    lanes = lax.iota(jnp.int32, L)
    dbufs = (dl0, dl1)
    wbufs = (wl0, wl1)

    def win_task(wt, _):
        win_id = wid * WPW + wt
        b = win_id // NWIN
        wi = win_id - b * NWIN
        src_off = b * N + wi * WIN
        pltpu.sync_copy(flow_hbm.at[pl.ds((b * 2 + 0) * N + wi * WIN, WIN)],
                        fxb)
        pltpu.sync_copy(flow_hbm.at[pl.ds((b * 2 + 1) * N + wi * WIN, WIN)],
                        fyb)
        pltpu.sync_copy(metric_hbm.at[pl.ds(src_off, WIN)], mb)
        y0row = wi * 4

        def vec(j, _):
            xi = (j % (W // L)) * L + lanes
            yrow = y0row + j // (W // L)
            fx = xi.astype(jnp.float32) + fxb[pl.ds(j * L, L)]
            fy = yrow.astype(jnp.float32) + fyb[pl.ds(j * L, L)]
            x0 = fx.astype(jnp.int32)
            x0 = jnp.where(x0.astype(jnp.float32) > fx, x0 - 1, x0)
            y0 = fy.astype(jnp.int32)
            y0 = jnp.where(y0.astype(jnp.float32) > fy, y0 - 1, y0)
            wx1 = fx - x0.astype(jnp.float32)
            wx0 = 1.0 - wx1
            wy1 = fy - y0.astype(jnp.float32)
            wy0 = 1.0 - wy1
            m = jnp.exp(mb[pl.ds(j * L, L)])
            x1 = x0 + 1
            y1 = y0 + 1
            vx0 = (x0 >= 0) & (x0 < W)
            vx1 = (x1 >= 0) & (x1 < W)
            vy0 = (y0 >= 0) & (y0 < H)
            vy1 = (y1 >= 0) & (y1 < H)
            cx0 = jnp.minimum(jnp.maximum(x0, 0), W - 1)
            cx1 = jnp.minimum(jnp.maximum(x1, 0), W - 1)
            cy0 = jnp.minimum(jnp.maximum(y0, 0), H - 1) * W
            cy1 = jnp.minimum(jnp.maximum(y1, 0), H - 1) * W
            zero = jnp.zeros((L,), jnp.float32)
            taps = (
                (cy0 + cx0, jnp.where(vx0 & vy0, wx0 * wy0 * m, zero)),
                (cy0 + cx1, jnp.where(vx1 & vy0, wx1 * wy0 * m, zero)),
                (cy1 + cx0, jnp.where(vx0 & vy1, wx0 * wy1 * m, zero)),
                (cy1 + cx1, jnp.where(vx1 & vy1, wx1 * wy1 * m, zero)),
            )
            zi = jnp.zeros((L,), jnp.int32)
            for half in range(2):
                base = half * HALF
                for t in range(4):
                    d, wv = taps[t]
                    local = d - base
                    inb = (local >= 0) & (local < HALF)
                    dbufs[half][pl.ds(t * WIN + j * L, L)] = (
                        jnp.where(inb, local, zi))
                    wbufs[half][pl.ds(t * WIN + j * L, L)] = (
                        jnp.where(inb, wv, zero))
            return 0

        lax.fori_loop(0, WIN // L, vec, 0)
        for half in range(2):
            off = (((b * 2 + half) * NWIN + wi) * 4) * WIN
            pltpu.sync_copy(dbufs[half], dst_hbm.at[pl.ds(off, 4 * WIN)])
            pltpu.sync_copy(wbufs[half], wgt_hbm.at[pl.ds(off, 4 * WIN)])
        return 0

    lax.fori_loop(0, WPW, win_task, 0)


def _pass2_body(dst_hbm, wgt_hbm, val_hbm, out_hbm,
                acc, db, wb, vb, sem0, sem1):
    wid = lax.axis_index("s") * NC + lax.axis_index("c")
    sems = (sem0, sem1)
    zero16 = jnp.zeros((L,), jnp.float32)

    def copies(slot, b, half, ch_flat, wi):
        toff = (((b * 2 + half) * NWIN + wi) * 4) * WIN
        return (
            pltpu.make_async_copy(dst_hbm.at[pl.ds(toff, 4 * WIN)],
                                  db.at[slot], sems[slot]),
            pltpu.make_async_copy(wgt_hbm.at[pl.ds(toff, 4 * WIN)],
                                  wb.at[slot], sems[slot]),
            pltpu.make_async_copy(val_hbm.at[pl.ds(ch_flat * N + wi * WIN,
                                                   WIN)],
                                  vb.at[slot], sems[slot]),
        )

    def task(k, _):
        tid = k * NW + wid

        @pl.when(tid < NTASK)
        def _():
            b = tid // (2 * (C + 1))
            rem = tid - b * (2 * (C + 1))
            ch = rem // 2
            half = rem - ch * 2
            ch_flat = b * (C + 1) + ch

            def zacc(z, _):
                acc[pl.ds(z * L, L)] = zero16
                return 0

            lax.fori_loop(0, HALF // L, zacc, 0)

            for cp in copies(0, b, half, ch_flat, 0):
                cp.start()

            def group(g, _):
                for s in range(2):
                    wi = g * 2 + s
                    nxt = wi + 1

                    @pl.when(nxt < NWIN)
                    def _():
                        for cp in copies(1 - s, b, half, ch_flat, nxt):
                            cp.start()

                    for cp in copies(s, b, half, ch_flat, wi):
                        cp.wait()

                    def vec(i, _):
                        v = vb[s, pl.ds(i * L, L)]
                        for t in range(4):
                            d = db[s, pl.ds(t * WIN + i * L, L)]
                            wv = wb[s, pl.ds(t * WIN + i * L, L)]
                            plsc.addupdate_scatter(acc, [d], wv * v)
                        return 0

                    lax.fori_loop(0, WIN // L, vec, 0)
                return 0

            lax.fori_loop(0, NWIN // 2, group, 0)
            pltpu.sync_copy(acc,
                            out_hbm.at[pl.ds(ch_flat * N + half * HALF,
                                             HALF)])

        return 0

    lax.fori_loop(0, KMAX, task, 0)


def _norm_body(num_ref, den_ref, o_ref):
    o_ref[...] = num_ref[...] / (den_ref[...] + 1e-7)


def kernel(tenInput, tenFlow, tenMetric):
    mesh = plsc.VectorSubcoreMesh(core_axis_name="c", subcore_axis_name="s")

    flow_flat = tenFlow.reshape(B * 2 * N)
    metric_flat = tenMetric.reshape(B * N)
    ones = jnp.ones((B, 1, H, W), dtype=tenInput.dtype)
    val_flat = jnp.concatenate([tenInput, ones], axis=1).reshape(B * (C + 1) * N)

    sc_params = pltpu.CompilerParams(needs_layout_passes=False)
    pass1 = functools.partial(
        pl.kernel,
        mesh=mesh,
        compiler_params=sc_params,
        out_type=(
            jax.ShapeDtypeStruct((B * 2 * 4 * N,), jnp.int32),
            jax.ShapeDtypeStruct((B * 2 * 4 * N,), jnp.float32),
        ),
        scratch_types=[
            pltpu.VMEM((WIN,), jnp.float32),
            pltpu.VMEM((WIN,), jnp.float32),
            pltpu.VMEM((WIN,), jnp.float32),
            pltpu.VMEM((4 * WIN,), jnp.int32),
            pltpu.VMEM((4 * WIN,), jnp.int32),
            pltpu.VMEM((4 * WIN,), jnp.float32),
            pltpu.VMEM((4 * WIN,), jnp.float32),
        ],
    )(_pass1_body)
    dst_flat, wgt_flat = pass1(flow_flat, metric_flat)

    pass2 = functools.partial(
        pl.kernel,
        mesh=mesh,
        compiler_params=sc_params,
        out_type=jax.ShapeDtypeStruct((B * (C + 1) * N,), jnp.float32),
        scratch_types=[
            pltpu.VMEM((HALF,), jnp.float32),
            pltpu.VMEM((2, 4 * WIN), jnp.int32),
            pltpu.VMEM((2, 4 * WIN), jnp.float32),
            pltpu.VMEM((2, WIN), jnp.float32),
            pltpu.SemaphoreType.DMA,
            pltpu.SemaphoreType.DMA,
        ],
    )(_pass2_body)
    out97 = pass2(dst_flat, wgt_flat, val_flat).reshape(B, C + 1, H, W)

    num = out97[:, :C]
    den = out97[:, C:]
    out = pl.pallas_call(
        _norm_body,
        grid=(B, C),
        in_specs=[
            pl.BlockSpec((1, 1, H, W), lambda b, c: (b, c, 0, 0)),
            pl.BlockSpec((1, 1, H, W), lambda b, c: (b, 0, 0, 0)),
        ],
        out_specs=pl.BlockSpec((1, 1, H, W), lambda b, c: (b, c, 0, 0)),
        out_shape=jax.ShapeDtypeStruct((B, C, H, W), jnp.float32),
    )(num, den)
    return out


# speedup vs baseline: 1.2369x; 1.2369x over previous
import functools

import jax
import jax.numpy as jnp
from jax import lax
from jax.experimental import pallas as pl
from jax.experimental.pallas import tpu as pltpu
from jax.experimental.pallas import tpu_sc as plsc

B = 2
C = 96
H = 384
W = 384
N = H * W
NC = 2
NS = 16
NW = NC * NS
HALF = N // 2
NTASK = B * (C + 1) * 2
KMAX = (NTASK + NW - 1) // NW
WIN = 4 * W
NWIN = N // WIN
WPW = (B * NWIN) // NW
L = 16


def _pass1_body(flow_hbm, metric_hbm, dst_hbm, wgt_hbm,
                fxb, fyb, mb, dl0, dl1, wl0, wl1):
    wid = lax.axis_index("s") * NC + lax.axis_index("c")
    lanes = lax.iota(jnp.int32, L)
    dbufs = (dl0, dl1)
    wbufs = (wl0, wl1)

    def win_task(wt, _):
        win_id = wid * WPW + wt
        b = win_id // NWIN
        wi = win_id - b * NWIN
        src_off = b * N + wi * WIN
        pltpu.sync_copy(flow_hbm.at[pl.ds((b * 2 + 0) * N + wi * WIN, WIN)],
                        fxb)
        pltpu.sync_copy(flow_hbm.at[pl.ds((b * 2 + 1) * N + wi * WIN, WIN)],
                        fyb)
        pltpu.sync_copy(metric_hbm.at[pl.ds(src_off, WIN)], mb)
        y0row = wi * 4

        def vec(j, _):
            xi = (j % (W // L)) * L + lanes
            yrow = y0row + j // (W // L)
            fx = xi.astype(jnp.float32) + fxb[pl.ds(j * L, L)]
            fy = yrow.astype(jnp.float32) + fyb[pl.ds(j * L, L)]
            x0 = fx.astype(jnp.int32)
            x0 = jnp.where(x0.astype(jnp.float32) > fx, x0 - 1, x0)
            y0 = fy.astype(jnp.int32)
            y0 = jnp.where(y0.astype(jnp.float32) > fy, y0 - 1, y0)
            wx1 = fx - x0.astype(jnp.float32)
            wx0 = 1.0 - wx1
            wy1 = fy - y0.astype(jnp.float32)
            wy0 = 1.0 - wy1
            m = jnp.exp(mb[pl.ds(j * L, L)])
            x1 = x0 + 1
            y1 = y0 + 1
            vx0 = (x0 >= 0) & (x0 < W)
            vx1 = (x1 >= 0) & (x1 < W)
            vy0 = (y0 >= 0) & (y0 < H)
            vy1 = (y1 >= 0) & (y1 < H)
            cx0 = jnp.minimum(jnp.maximum(x0, 0), W - 1)
            cx1 = jnp.minimum(jnp.maximum(x1, 0), W - 1)
            cy0 = jnp.minimum(jnp.maximum(y0, 0), H - 1) * W
            cy1 = jnp.minimum(jnp.maximum(y1, 0), H - 1) * W
            zero = jnp.zeros((L,), jnp.float32)
            taps = (
                (cy0 + cx0, jnp.where(vx0 & vy0, wx0 * wy0 * m, zero)),
                (cy0 + cx1, jnp.where(vx1 & vy0, wx1 * wy0 * m, zero)),
                (cy1 + cx0, jnp.where(vx0 & vy1, wx0 * wy1 * m, zero)),
                (cy1 + cx1, jnp.where(vx1 & vy1, wx1 * wy1 * m, zero)),
            )
            zi = jnp.zeros((L,), jnp.int32)
            for half in range(2):
                base = half * HALF
                for t in range(4):
                    d, wv = taps[t]
                    local = d - base
                    inb = (local >= 0) & (local < HALF)
                    dbufs[half][pl.ds(t * WIN + j * L, L)] = (
                        jnp.where(inb, local, zi))
                    wbufs[half][pl.ds(t * WIN + j * L, L)] = (
                        jnp.where(inb, wv, zero))
            return 0

        lax.fori_loop(0, WIN // L, vec, 0)
        for half in range(2):
            off = (((b * 2 + half) * NWIN + wi) * 4) * WIN
            pltpu.sync_copy(dbufs[half], dst_hbm.at[pl.ds(off, 4 * WIN)])
            pltpu.sync_copy(wbufs[half], wgt_hbm.at[pl.ds(off, 4 * WIN)])
        return 0

    lax.fori_loop(0, WPW, win_task, 0)


def _pass2_body(dst_hbm, wgt_hbm, val_hbm, out_hbm,
                acc, db, wb, vb, sem0, sem1):
    wid = lax.axis_index("s") * NC + lax.axis_index("c")
    sems = (sem0, sem1)
    zero16 = jnp.zeros((L,), jnp.float32)

    def copies(slot, b, half, ch_flat, wi):
        toff = (((b * 2 + half) * NWIN + wi) * 4) * WIN
        return (
            pltpu.make_async_copy(dst_hbm.at[pl.ds(toff, 4 * WIN)],
                                  db.at[slot], sems[slot]),
            pltpu.make_async_copy(wgt_hbm.at[pl.ds(toff, 4 * WIN)],
                                  wb.at[slot], sems[slot]),
            pltpu.make_async_copy(val_hbm.at[pl.ds(ch_flat * N + wi * WIN,
                                                   WIN)],
                                  vb.at[slot], sems[slot]),
        )

    def task(k, _):
        tid = k * NW + wid

        @pl.when(tid < NTASK)
        def _():
            b = tid // (2 * (C + 1))
            rem = tid - b * (2 * (C + 1))
            ch = rem // 2
            half = rem - ch * 2
            ch_flat = b * (C + 1) + ch

            def zacc(z, _):
                acc[pl.ds(z * L, L)] = zero16
                return 0

            lax.fori_loop(0, HALF // L, zacc, 0)

            for cp in copies(0, b, half, ch_flat, 0):
                cp.start()

            def group(g, _):
                for s in range(2):
                    wi = g * 2 + s
                    nxt = wi + 1

                    @pl.when(nxt < NWIN)
                    def _():
                        for cp in copies(1 - s, b, half, ch_flat, nxt):
                            cp.start()

                    for cp in copies(s, b, half, ch_flat, wi):
                        cp.wait()

                    @plsc.parallel_loop(0, WIN // L, unroll=4)
                    def vec(i):
                        v = vb[s, pl.ds(i * L, L)]
                        for t in range(4):
                            d = db[s, pl.ds(t * WIN + i * L, L)]
                            wv = wb[s, pl.ds(t * WIN + i * L, L)]
                            plsc.addupdate_scatter(acc, [d], wv * v)
                return 0

            lax.fori_loop(0, NWIN // 2, group, 0)
            pltpu.sync_copy(acc,
                            out_hbm.at[pl.ds(ch_flat * N + half * HALF,
                                             HALF)])

        return 0

    lax.fori_loop(0, KMAX, task, 0)


def _norm_body(num_ref, den_ref, o_ref):
    o_ref[...] = num_ref[...] / (den_ref[...] + 1e-7)


def kernel(tenInput, tenFlow, tenMetric):
    mesh = plsc.VectorSubcoreMesh(core_axis_name="c", subcore_axis_name="s")

    flow_flat = tenFlow.reshape(B * 2 * N)
    metric_flat = tenMetric.reshape(B * N)
    ones = jnp.ones((B, 1, H, W), dtype=tenInput.dtype)
    val_flat = jnp.concatenate([tenInput, ones], axis=1).reshape(B * (C + 1) * N)

    sc_params = pltpu.CompilerParams(needs_layout_passes=False)
    pass1 = functools.partial(
        pl.kernel,
        mesh=mesh,
        compiler_params=sc_params,
        out_type=(
            jax.ShapeDtypeStruct((B * 2 * 4 * N,), jnp.int32),
            jax.ShapeDtypeStruct((B * 2 * 4 * N,), jnp.float32),
        ),
        scratch_types=[
            pltpu.VMEM((WIN,), jnp.float32),
            pltpu.VMEM((WIN,), jnp.float32),
            pltpu.VMEM((WIN,), jnp.float32),
            pltpu.VMEM((4 * WIN,), jnp.int32),
            pltpu.VMEM((4 * WIN,), jnp.int32),
            pltpu.VMEM((4 * WIN,), jnp.float32),
            pltpu.VMEM((4 * WIN,), jnp.float32),
        ],
    )(_pass1_body)
    dst_flat, wgt_flat = pass1(flow_flat, metric_flat)

    pass2 = functools.partial(
        pl.kernel,
        mesh=mesh,
        compiler_params=sc_params,
        out_type=jax.ShapeDtypeStruct((B * (C + 1) * N,), jnp.float32),
        scratch_types=[
            pltpu.VMEM((HALF,), jnp.float32),
            pltpu.VMEM((2, 4 * WIN), jnp.int32),
            pltpu.VMEM((2, 4 * WIN), jnp.float32),
            pltpu.VMEM((2, WIN), jnp.float32),
            pltpu.SemaphoreType.DMA,
            pltpu.SemaphoreType.DMA,
        ],
    )(_pass2_body)
    out97 = pass2(dst_flat, wgt_flat, val_flat).reshape(B, C + 1, H, W)

    num = out97[:, :C]
    den = out97[:, C:]
    out = pl.pallas_call(
        _norm_body,
        grid=(B, C),
        in_specs=[
            pl.BlockSpec((1, 1, H, W), lambda b, c: (b, c, 0, 0)),
            pl.BlockSpec((1, 1, H, W), lambda b, c: (b, 0, 0, 0)),
        ],
        out_specs=pl.BlockSpec((1, 1, H, W), lambda b, c: (b, c, 0, 0)),
        out_shape=jax.ShapeDtypeStruct((B, C, H, W), jnp.float32),
    )(num, den)
    return out


# speedup vs baseline: 2.6468x; 2.1398x over previous
import functools

import jax
import jax.numpy as jnp
from jax import lax
from jax.experimental import pallas as pl
from jax.experimental.pallas import tpu as pltpu
from jax.experimental.pallas import tpu_sc as plsc

B = 2
C = 96
H = 384
W = 384
N = H * W
NC = 2
NS = 16
NW = NC * NS
HALF = N // 2
NTASK = B * (C + 1) * 2
KMAX = (NTASK + NW - 1) // NW
WIN = 4 * W
NWIN = N // WIN
WPW = (B * NWIN) // NW
L = 16


def _pass1_body(flow_hbm, metric_hbm, dst_hbm, wgt_hbm,
                fxb, fyb, mb, dl0, dl1, wl0, wl1):
    wid = lax.axis_index("s") * NC + lax.axis_index("c")
    lanes = lax.iota(jnp.int32, L)
    dbufs = (dl0, dl1)
    wbufs = (wl0, wl1)

    def win_task(wt, _):
        win_id = wid * WPW + wt
        b = win_id // NWIN
        wi = win_id - b * NWIN
        src_off = b * N + wi * WIN
        pltpu.sync_copy(flow_hbm.at[pl.ds((b * 2 + 0) * N + wi * WIN, WIN)],
                        fxb)
        pltpu.sync_copy(flow_hbm.at[pl.ds((b * 2 + 1) * N + wi * WIN, WIN)],
                        fyb)
        pltpu.sync_copy(metric_hbm.at[pl.ds(src_off, WIN)], mb)
        y0row = wi * 4

        def vec(j, _):
            xi = (j % (W // L)) * L + lanes
            yrow = y0row + j // (W // L)
            fx = xi.astype(jnp.float32) + fxb[pl.ds(j * L, L)]
            fy = yrow.astype(jnp.float32) + fyb[pl.ds(j * L, L)]
            x0 = fx.astype(jnp.int32)
            x0 = jnp.where(x0.astype(jnp.float32) > fx, x0 - 1, x0)
            y0 = fy.astype(jnp.int32)
            y0 = jnp.where(y0.astype(jnp.float32) > fy, y0 - 1, y0)
            wx1 = fx - x0.astype(jnp.float32)
            wx0 = 1.0 - wx1
            wy1 = fy - y0.astype(jnp.float32)
            wy0 = 1.0 - wy1
            m = jnp.exp(mb[pl.ds(j * L, L)])
            x1 = x0 + 1
            y1 = y0 + 1
            vx0 = (x0 >= 0) & (x0 < W)
            vx1 = (x1 >= 0) & (x1 < W)
            vy0 = (y0 >= 0) & (y0 < H)
            vy1 = (y1 >= 0) & (y1 < H)
            cx0 = jnp.minimum(jnp.maximum(x0, 0), W - 1)
            cx1 = jnp.minimum(jnp.maximum(x1, 0), W - 1)
            cy0 = jnp.minimum(jnp.maximum(y0, 0), H - 1) * W
            cy1 = jnp.minimum(jnp.maximum(y1, 0), H - 1) * W
            zero = jnp.zeros((L,), jnp.float32)
            taps = (
                (cy0 + cx0, jnp.where(vx0 & vy0, wx0 * wy0 * m, zero)),
                (cy0 + cx1, jnp.where(vx1 & vy0, wx1 * wy0 * m, zero)),
                (cy1 + cx0, jnp.where(vx0 & vy1, wx0 * wy1 * m, zero)),
                (cy1 + cx1, jnp.where(vx1 & vy1, wx1 * wy1 * m, zero)),
            )
            zi = jnp.zeros((L,), jnp.int32)
            for half in range(2):
                base = half * HALF
                for t in range(4):
                    d, wv = taps[t]
                    local = d - base
                    inb = (local >= 0) & (local < HALF)
                    dbufs[half][pl.ds(t * WIN + j * L, L)] = (
                        jnp.where(inb, local, zi))
                    wbufs[half][pl.ds(t * WIN + j * L, L)] = (
                        jnp.where(inb, wv, zero))
            return 0

        lax.fori_loop(0, WIN // L, vec, 0)
        for half in range(2):
            off = (((b * 2 + half) * NWIN + wi) * 4) * WIN
            pltpu.sync_copy(dbufs[half], dst_hbm.at[pl.ds(off, 4 * WIN)])
            pltpu.sync_copy(wbufs[half], wgt_hbm.at[pl.ds(off, 4 * WIN)])
        return 0

    lax.fori_loop(0, WPW, win_task, 0)


def _pass2_body(dst_hbm, wgt_hbm, val_hbm, out_hbm,
                acc, db, wb, vb, sem0, sem1):
    wid = lax.axis_index("s") * NC + lax.axis_index("c")
    sems = (sem0, sem1)
    zero16 = jnp.zeros((L,), jnp.float32)

    def copies(slot, b, half, ch_flat, wi):
        toff = (((b * 2 + half) * NWIN + wi) * 4) * WIN
        return (
            pltpu.make_async_copy(dst_hbm.at[pl.ds(toff, 4 * WIN)],
                                  db.at[slot], sems[slot]),
            pltpu.make_async_copy(wgt_hbm.at[pl.ds(toff, 4 * WIN)],
                                  wb.at[slot], sems[slot]),
            pltpu.make_async_copy(val_hbm.at[pl.ds(ch_flat * N + wi * WIN,
                                                   WIN)],
                                  vb.at[slot], sems[slot]),
        )

    def task(k, _):
        tid = k * NW + wid

        @pl.when(tid < NTASK)
        def _():
            b = tid // (2 * (C + 1))
            rem = tid - b * (2 * (C + 1))
            ch = rem // 2
            half = rem - ch * 2
            ch_flat = b * (C + 1) + ch

            def zacc(z, _):
                acc[pl.ds(z * L, L)] = zero16
                return 0

            lax.fori_loop(0, HALF // L, zacc, 0)

            for cp in copies(0, b, half, ch_flat, 0):
                cp.start()

            def group(g, _):
                for s in range(2):
                    wi = g * 2 + s
                    nxt = wi + 1

                    @pl.when(nxt < NWIN)
                    def _():
                        for cp in copies(1 - s, b, half, ch_flat, nxt):
                            cp.start()

                    for cp in copies(s, b, half, ch_flat, wi):
                        cp.wait()

                    @plsc.parallel_loop(0, WIN // L, unroll=4)
                    def vec(i):
                        v = vb[s, pl.ds(i * L, L)]
                        for t in range(4):
                            d = db[s, pl.ds(t * WIN + i * L, L)]
                            wv = wb[s, pl.ds(t * WIN + i * L, L)]

                            @pl.when(jnp.any(wv != 0.0))
                            def _():
                                plsc.addupdate_scatter(acc, [d], wv * v)
                return 0

            lax.fori_loop(0, NWIN // 2, group, 0)
            pltpu.sync_copy(acc,
                            out_hbm.at[pl.ds(ch_flat * N + half * HALF,
                                             HALF)])

        return 0

    lax.fori_loop(0, KMAX, task, 0)


def _norm_body(num_ref, den_ref, o_ref):
    o_ref[...] = num_ref[...] / (den_ref[...] + 1e-7)


def kernel(tenInput, tenFlow, tenMetric):
    mesh = plsc.VectorSubcoreMesh(core_axis_name="c", subcore_axis_name="s")

    flow_flat = tenFlow.reshape(B * 2 * N)
    metric_flat = tenMetric.reshape(B * N)
    ones = jnp.ones((B, 1, H, W), dtype=tenInput.dtype)
    val_flat = jnp.concatenate([tenInput, ones], axis=1).reshape(B * (C + 1) * N)

    sc_params = pltpu.CompilerParams(needs_layout_passes=False)
    pass1 = functools.partial(
        pl.kernel,
        mesh=mesh,
        compiler_params=sc_params,
        out_type=(
            jax.ShapeDtypeStruct((B * 2 * 4 * N,), jnp.int32),
            jax.ShapeDtypeStruct((B * 2 * 4 * N,), jnp.float32),
        ),
        scratch_types=[
            pltpu.VMEM((WIN,), jnp.float32),
            pltpu.VMEM((WIN,), jnp.float32),
            pltpu.VMEM((WIN,), jnp.float32),
            pltpu.VMEM((4 * WIN,), jnp.int32),
            pltpu.VMEM((4 * WIN,), jnp.int32),
            pltpu.VMEM((4 * WIN,), jnp.float32),
            pltpu.VMEM((4 * WIN,), jnp.float32),
        ],
    )(_pass1_body)
    dst_flat, wgt_flat = pass1(flow_flat, metric_flat)

    pass2 = functools.partial(
        pl.kernel,
        mesh=mesh,
        compiler_params=sc_params,
        out_type=jax.ShapeDtypeStruct((B * (C + 1) * N,), jnp.float32),
        scratch_types=[
            pltpu.VMEM((HALF,), jnp.float32),
            pltpu.VMEM((2, 4 * WIN), jnp.int32),
            pltpu.VMEM((2, 4 * WIN), jnp.float32),
            pltpu.VMEM((2, WIN), jnp.float32),
            pltpu.SemaphoreType.DMA,
            pltpu.SemaphoreType.DMA,
        ],
    )(_pass2_body)
    out97 = pass2(dst_flat, wgt_flat, val_flat).reshape(B, C + 1, H, W)

    num = out97[:, :C]
    den = out97[:, C:]
    out = pl.pallas_call(
        _norm_body,
        grid=(B, C),
        in_specs=[
            pl.BlockSpec((1, 1, H, W), lambda b, c: (b, c, 0, 0)),
            pl.BlockSpec((1, 1, H, W), lambda b, c: (b, 0, 0, 0)),
        ],
        out_specs=pl.BlockSpec((1, 1, H, W), lambda b, c: (b, c, 0, 0)),
        out_shape=jax.ShapeDtypeStruct((B, C, H, W), jnp.float32),
    )(num, den)
    return out
